# Initial kernel scaffold; baseline (speedup 1.0000x reference)
import jax
import jax.numpy as jnp
from jax.experimental import pallas as pl


def kernel(xyz, new_xyz, components, new_components, features):
    raise NotImplementedError("write your pallas kernel here")


# speedup vs baseline: 131.5217x; 131.5217x over previous
import functools

import jax
import jax.numpy as jnp
from jax import lax
from jax.experimental import pallas as pl
from jax.experimental.pallas import tpu as pltpu
from jax.experimental.pallas import tpu_sc as plsc

_B, _N, _P, _C, _S = 4, 8192, 2048, 64, 32
_R2 = 0.2 * 0.2
_R2P = (0.5 * 0.2) ** 2
_NW = 32
_WPB = _NW // _B
_QPW = _P // _WPB
_WPC = _C // _WPB
_CH = 4096

_mesh = plsc.VectorSubcoreMesh(core_axis_name="c", subcore_axis_name="s")


def _wid():
    return lax.axis_index("s") * 2 + lax.axis_index("c")


@functools.partial(
    pl.kernel,
    out_type=(
        jax.ShapeDtypeStruct((_B * _P * _S,), jnp.int32),
        jax.ShapeDtypeStruct((_B * _P * _S,), jnp.float32),
        jax.ShapeDtypeStruct((_B * _P * _S,), jnp.float32),
        jax.ShapeDtypeStruct((_B * _P * _S,), jnp.float32),
    ),
    mesh=_mesh,
    compiler_params=pltpu.CompilerParams(needs_layout_passes=False),
    scratch_types=[
        pltpu.VMEM((_N,), jnp.float32),
        pltpu.VMEM((_N,), jnp.float32),
        pltpu.VMEM((_N,), jnp.float32),
        pltpu.VMEM((_N,), jnp.int32),
        pltpu.VMEM((_QPW,), jnp.float32),
        pltpu.VMEM((_QPW,), jnp.float32),
        pltpu.VMEM((_QPW,), jnp.float32),
        pltpu.VMEM((_QPW,), jnp.int32),
        pltpu.VMEM((_QPW * _S + 16,), jnp.int32),
        pltpu.VMEM((_QPW * _S,), jnp.float32),
        pltpu.VMEM((_QPW * _S,), jnp.float32),
        pltpu.VMEM((_QPW * _S,), jnp.float32),
    ],
)
def _phase1(xs_hbm, ys_hbm, zs_hbm, comp_hbm, nxs_hbm, nys_hbm, nzs_hbm,
            ncomp_hbm, idx_out, gx_out, gy_out, gz_out,
            ptx, pty, ptz, ptc, qxv, qyv, qzv, qcv,
            st_idx, st_dx, st_dy, st_dz):
    w = _wid()
    b = w // _WPB
    q0 = b * _P + (w % _WPB) * _QPW
    pltpu.sync_copy(xs_hbm.at[pl.ds(b * _N, _N)], ptx)
    pltpu.sync_copy(ys_hbm.at[pl.ds(b * _N, _N)], pty)
    pltpu.sync_copy(zs_hbm.at[pl.ds(b * _N, _N)], ptz)
    pltpu.sync_copy(comp_hbm.at[pl.ds(b * _N, _N)], ptc)
    pltpu.sync_copy(nxs_hbm.at[pl.ds(q0, _QPW)], qxv)
    pltpu.sync_copy(nys_hbm.at[pl.ds(q0, _QPW)], qyv)
    pltpu.sync_copy(nzs_hbm.at[pl.ds(q0, _QPW)], qzv)
    pltpu.sync_copy(ncomp_hbm.at[pl.ds(q0, _QPW)], qcv)

    lane = lax.iota(jnp.int32, 16)
    zeros16 = jnp.zeros((16,), jnp.int32)
    unroll = 4
    smax = jnp.full((16,), _S, jnp.int32)

    def per_query(p, carry):
        base = p * _S
        spl = jnp.full((16,), p, jnp.int32)
        qx = plsc.load_gather(qxv, [spl])
        qy = plsc.load_gather(qyv, [spl])
        qz = plsc.load_gather(qzv, [spl])
        qc = plsc.load_gather(qcv, [spl])
        base_v = jnp.full((16,), base, jnp.int32)
        st_idx[pl.ds(base, 16)] = zeros16

        def chunk(j, cnt):
            for u in range(unroll):
                off = j * (16 * unroll) + u * 16
                px = ptx[pl.ds(off, 16)]
                py = pty[pl.ds(off, 16)]
                pz = ptz[pl.ds(off, 16)]
                pc = ptc[pl.ds(off, 16)]
                dx = px - qx
                dy = py - qy
                dz = pz - qz
                d2 = dx * dx + dy * dy + dz * dz
                thr = jnp.where(pc == qc, _R2, _R2P).astype(jnp.float32)
                elig = d2 < thr
                ei = elig.astype(jnp.int32)
                excl = plsc.cumsum(ei) - ei
                pos = base_v + jnp.minimum(cnt, smax) + excl
                plsc.store_scatter(st_idx, [pos], lane + off, mask=elig)
                cnt = cnt + plsc.all_reduce_population_count(elig)
            return cnt

        cnt = lax.fori_loop(0, _N // (16 * unroll), chunk, zeros16)
        cntc = jnp.minimum(cnt, smax)
        pad = plsc.load_gather(st_idx, [base_v])
        for h in range(2):
            lids = lane + h * 16
            plsc.store_scatter(st_idx, [base_v + lids], pad, mask=lids >= cntc)
        for h in range(2):
            iv = st_idx[pl.ds(base + h * 16, 16)]
            st_dx[pl.ds(base + h * 16, 16)] = plsc.load_gather(ptx, [iv]) - qx
            st_dy[pl.ds(base + h * 16, 16)] = plsc.load_gather(pty, [iv]) - qy
            st_dz[pl.ds(base + h * 16, 16)] = plsc.load_gather(ptz, [iv]) - qz
        return carry

    lax.fori_loop(0, _QPW, per_query, 0)

    o0 = q0 * _S
    pltpu.sync_copy(st_idx.at[pl.ds(0, _QPW * _S)],
                    idx_out.at[pl.ds(o0, _QPW * _S)])
    pltpu.sync_copy(st_dx, gx_out.at[pl.ds(o0, _QPW * _S)])
    pltpu.sync_copy(st_dy, gy_out.at[pl.ds(o0, _QPW * _S)])
    pltpu.sync_copy(st_dz, gz_out.at[pl.ds(o0, _QPW * _S)])


@functools.partial(
    pl.kernel,
    out_type=jax.ShapeDtypeStruct((_B * _C * _P * _S,), jnp.float32),
    mesh=_mesh,
    compiler_params=pltpu.CompilerParams(needs_layout_passes=False),
    scratch_types=[
        pltpu.VMEM((_WPC * _N,), jnp.float32),
        pltpu.VMEM((_CH,), jnp.int32),
        pltpu.VMEM((_WPC * _CH,), jnp.float32),
    ],
)
def _phase2(feat_hbm, idx_hbm, gf_out, tabs, idxc, stg):
    w = _wid()
    b = w // _WPB
    c0 = (w % _WPB) * _WPC
    pltpu.sync_copy(feat_hbm.at[pl.ds((b * _C + c0) * _N, _WPC * _N)], tabs)
    chbase = [jnp.full((16,), ch * _N, jnp.int32) for ch in range(_WPC)]

    def per_chunk(k, carry):
        k0 = k * _CH
        pltpu.sync_copy(idx_hbm.at[pl.ds(b * _P * _S + k0, _CH)], idxc)

        def inner(t, c2):
            for u in range(4):
                o = t * 64 + u * 16
                iv = idxc[pl.ds(o, 16)]
                for ch in range(_WPC):
                    stg[pl.ds(ch * _CH + o, 16)] = plsc.load_gather(
                        tabs, [chbase[ch] + iv])
            return c2

        lax.fori_loop(0, _CH // 64, inner, 0)
        for ch in range(_WPC):
            pltpu.sync_copy(
                stg.at[pl.ds(ch * _CH, _CH)],
                gf_out.at[pl.ds((b * _C + c0 + ch) * (_P * _S) + k0, _CH)])
        return carry

    lax.fori_loop(0, (_P * _S) // _CH, per_chunk, 0)


def kernel(xyz, new_xyz, components, new_components, features):
    xyz32 = xyz.astype(jnp.float32)
    nxyz32 = new_xyz.astype(jnp.float32)
    xs = xyz32[:, :, 0].reshape(-1)
    ys = xyz32[:, :, 1].reshape(-1)
    zs = xyz32[:, :, 2].reshape(-1)
    nxs = nxyz32[:, :, 0].reshape(-1)
    nys = nxyz32[:, :, 1].reshape(-1)
    nzs = nxyz32[:, :, 2].reshape(-1)
    comp = components.reshape(-1).astype(jnp.int32)
    ncomp = new_components.reshape(-1).astype(jnp.int32)
    idx, gx, gy, gz = _phase1(xs, ys, zs, comp, nxs, nys, nzs, ncomp)
    gfeat = _phase2(features.astype(jnp.float32).reshape(-1), idx)
    gxyz = jnp.stack([gx.reshape(_B, _P, _S), gy.reshape(_B, _P, _S),
                      gz.reshape(_B, _P, _S)], axis=1)
    return jnp.concatenate(
        [gxyz, gfeat.reshape(_B, _C, _P, _S)], axis=1)


# speedup vs baseline: 214.6041x; 1.6317x over previous
import functools

import jax
import jax.numpy as jnp
from jax import lax
from jax.experimental import pallas as pl
from jax.experimental.pallas import tpu as pltpu
from jax.experimental.pallas import tpu_sc as plsc

_B, _N, _P, _C, _S = 4, 8192, 2048, 64, 32
_R2 = 0.2 * 0.2
_R2P = (0.5 * 0.2) ** 2
_NW = 32
_WPB = _NW // _B
_QPW = _P // _WPB
_WPC = _C // _WPB
_CH = 4096

_mesh = plsc.VectorSubcoreMesh(core_axis_name="c", subcore_axis_name="s")


def _wid():
    return lax.axis_index("s") * 2 + lax.axis_index("c")


@functools.partial(
    pl.kernel,
    out_type=(
        jax.ShapeDtypeStruct((_B * _P * _S,), jnp.int32),
        jax.ShapeDtypeStruct((_B * _P * _S,), jnp.float32),
        jax.ShapeDtypeStruct((_B * _P * _S,), jnp.float32),
        jax.ShapeDtypeStruct((_B * _P * _S,), jnp.float32),
    ),
    mesh=_mesh,
    compiler_params=pltpu.CompilerParams(needs_layout_passes=False),
    scratch_types=[
        pltpu.VMEM((_N,), jnp.float32),
        pltpu.VMEM((_N,), jnp.float32),
        pltpu.VMEM((_N,), jnp.float32),
        pltpu.VMEM((_N,), jnp.int32),
        pltpu.VMEM((_QPW,), jnp.float32),
        pltpu.VMEM((_QPW,), jnp.float32),
        pltpu.VMEM((_QPW,), jnp.float32),
        pltpu.VMEM((_QPW,), jnp.int32),
        pltpu.VMEM((_QPW * 64,), jnp.int32),
        pltpu.VMEM((_QPW * _S,), jnp.int32),
        pltpu.VMEM((_QPW * _S,), jnp.float32),
        pltpu.VMEM((_QPW * _S,), jnp.float32),
        pltpu.VMEM((_QPW * _S,), jnp.float32),
    ],
)
def _phase1(xs_hbm, ys_hbm, zs_hbm, comp_hbm, nxs_hbm, nys_hbm, nzs_hbm,
            ncomp_hbm, idx_out, gx_out, gy_out, gz_out,
            ptx, pty, ptz, ptc, qxv, qyv, qzv, qcv,
            st_idx, st_idxc, st_dx, st_dy, st_dz):
    w = _wid()
    b = w // _WPB
    q0 = b * _P + (w % _WPB) * _QPW
    pltpu.sync_copy(xs_hbm.at[pl.ds(b * _N, _N)], ptx)
    pltpu.sync_copy(ys_hbm.at[pl.ds(b * _N, _N)], pty)
    pltpu.sync_copy(zs_hbm.at[pl.ds(b * _N, _N)], ptz)
    pltpu.sync_copy(comp_hbm.at[pl.ds(b * _N, _N)], ptc)
    pltpu.sync_copy(nxs_hbm.at[pl.ds(q0, _QPW)], qxv)
    pltpu.sync_copy(nys_hbm.at[pl.ds(q0, _QPW)], qyv)
    pltpu.sync_copy(nzs_hbm.at[pl.ds(q0, _QPW)], qzv)
    pltpu.sync_copy(ncomp_hbm.at[pl.ds(q0, _QPW)], qcv)

    lane = lax.iota(jnp.int32, 16)
    zeros16 = jnp.zeros((16,), jnp.int32)
    unroll = 4
    smax = jnp.full((16,), _S, jnp.int32)

    ones16 = jnp.ones((16,), jnp.int32)
    half = _QPW // 2

    def per_query(p, carry):
        qs = (p, p + half)
        qxyzc = []
        bases = []
        for q in qs:
            spl = jnp.full((16,), q, jnp.int32)
            qxyzc.append((plsc.load_gather(qxv, [spl]),
                          plsc.load_gather(qyv, [spl]),
                          plsc.load_gather(qzv, [spl]),
                          plsc.load_gather(qcv, [spl])))
            sbase = q * 64
            bases.append((q * _S, jnp.full((16,), sbase, jnp.int32),
                          jnp.full((16,), sbase - 1, jnp.int32), sbase))
            st_idx[pl.ds(sbase, 16)] = zeros16

        def chunk(j, ccs):
            out = []
            for u in range(unroll):
                off = j * (16 * unroll) + u * 16
                px = ptx[pl.ds(off, 16)]
                py = pty[pl.ds(off, 16)]
                pz = ptz[pl.ds(off, 16)]
                pc = ptc[pl.ds(off, 16)]
                gidx = lane + off
                out = []
                for (qx, qy, qz, qc), (_, base_v, bm1_v, _sb), cc in zip(
                        qxyzc, bases, ccs):
                    dx = px - qx
                    dy = py - qy
                    dz = pz - qz
                    d2 = dx * dx + dy * dy + dz * dz
                    thr = jnp.where(pc == qc, _R2, _R2P).astype(jnp.float32)
                    elig = d2 < thr
                    incl = plsc.cumsum(elig.astype(jnp.int32))
                    pos = (bm1_v + cc) + incl
                    plsc.store_scatter(st_idx, [pos], gidx, mask=elig)
                    cc = jnp.minimum(
                        cc + plsc.all_reduce_population_count(elig), smax)
                    out.append(cc)
                ccs = tuple(out)
            return ccs

        ccs = lax.fori_loop(0, _N // (16 * unroll), chunk,
                            (zeros16, zeros16))
        for (base, base_v, _b1, sbase), (qx, qy, qz, qc), cntc in zip(
                bases, qxyzc, ccs):
            pad = plsc.load_gather(st_idx, [base_v])
            for h in range(2):
                lids = lane + h * 16
                iv = jnp.where(lids >= cntc,
                               pad, st_idx[pl.ds(sbase + h * 16, 16)])
                st_idxc[pl.ds(base + h * 16, 16)] = iv
                st_dx[pl.ds(base + h * 16, 16)] = plsc.load_gather(ptx, [iv]) - qx
                st_dy[pl.ds(base + h * 16, 16)] = plsc.load_gather(pty, [iv]) - qy
                st_dz[pl.ds(base + h * 16, 16)] = plsc.load_gather(ptz, [iv]) - qz
        return carry

    lax.fori_loop(0, half, per_query, 0)

    o0 = q0 * _S
    pltpu.sync_copy(st_idxc, idx_out.at[pl.ds(o0, _QPW * _S)])
    pltpu.sync_copy(st_dx, gx_out.at[pl.ds(o0, _QPW * _S)])
    pltpu.sync_copy(st_dy, gy_out.at[pl.ds(o0, _QPW * _S)])
    pltpu.sync_copy(st_dz, gz_out.at[pl.ds(o0, _QPW * _S)])


@functools.partial(
    pl.kernel,
    out_type=jax.ShapeDtypeStruct((_B * _C * _P * _S,), jnp.float32),
    mesh=_mesh,
    compiler_params=pltpu.CompilerParams(needs_layout_passes=False),
    scratch_types=[
        pltpu.VMEM((_WPC * _N,), jnp.float32),
        pltpu.VMEM((_CH,), jnp.int32),
        pltpu.VMEM((_WPC * _CH,), jnp.float32),
    ],
)
def _phase2(feat_hbm, idx_hbm, gf_out, tabs, idxc, stg):
    w = _wid()
    b = w // _WPB
    c0 = (w % _WPB) * _WPC
    pltpu.sync_copy(feat_hbm.at[pl.ds((b * _C + c0) * _N, _WPC * _N)], tabs)
    chbase = [jnp.full((16,), ch * _N, jnp.int32) for ch in range(_WPC)]

    def per_chunk(k, carry):
        k0 = k * _CH
        pltpu.sync_copy(idx_hbm.at[pl.ds(b * _P * _S + k0, _CH)], idxc)

        def inner(t, c2):
            for u in range(4):
                o = t * 64 + u * 16
                iv = idxc[pl.ds(o, 16)]
                for ch in range(_WPC):
                    stg[pl.ds(ch * _CH + o, 16)] = plsc.load_gather(
                        tabs, [chbase[ch] + iv])
            return c2

        lax.fori_loop(0, _CH // 64, inner, 0)
        for ch in range(_WPC):
            pltpu.sync_copy(
                stg.at[pl.ds(ch * _CH, _CH)],
                gf_out.at[pl.ds((b * _C + c0 + ch) * (_P * _S) + k0, _CH)])
        return carry

    lax.fori_loop(0, (_P * _S) // _CH, per_chunk, 0)


def kernel(xyz, new_xyz, components, new_components, features):
    xyz32 = xyz.astype(jnp.float32)
    nxyz32 = new_xyz.astype(jnp.float32)
    xs = xyz32[:, :, 0].reshape(-1)
    ys = xyz32[:, :, 1].reshape(-1)
    zs = xyz32[:, :, 2].reshape(-1)
    nxs = nxyz32[:, :, 0].reshape(-1)
    nys = nxyz32[:, :, 1].reshape(-1)
    nzs = nxyz32[:, :, 2].reshape(-1)
    comp = components.reshape(-1).astype(jnp.int32)
    ncomp = new_components.reshape(-1).astype(jnp.int32)
    idx, gx, gy, gz = _phase1(xs, ys, zs, comp, nxs, nys, nzs, ncomp)
    gfeat = _phase2(features.astype(jnp.float32).reshape(-1), idx)
    gxyz = jnp.stack([gx.reshape(_B, _P, _S), gy.reshape(_B, _P, _S),
                      gz.reshape(_B, _P, _S)], axis=1)
    return jnp.concatenate(
        [gxyz, gfeat.reshape(_B, _C, _P, _S)], axis=1)


# speedup vs baseline: 283.6987x; 1.3220x over previous
import functools

import jax
import jax.numpy as jnp
from jax import lax
from jax.experimental import pallas as pl
from jax.experimental.pallas import tpu as pltpu
from jax.experimental.pallas import tpu_sc as plsc

_B, _N, _P, _C, _S = 4, 8192, 2048, 64, 32
_R2 = 0.2 * 0.2
_R2P = (0.5 * 0.2) ** 2
_NW = 32
_WPB = _NW // _B
_QPW = _P // _WPB
_WPC = _C // _WPB
_CH = 4096

_mesh = plsc.VectorSubcoreMesh(core_axis_name="c", subcore_axis_name="s")


def _wid():
    return lax.axis_index("s") * 2 + lax.axis_index("c")


@functools.partial(
    pl.kernel,
    out_type=(
        jax.ShapeDtypeStruct((_B * _P * _S,), jnp.int32),
        jax.ShapeDtypeStruct((_B * _P * _S,), jnp.float32),
        jax.ShapeDtypeStruct((_B * _P * _S,), jnp.float32),
        jax.ShapeDtypeStruct((_B * _P * _S,), jnp.float32),
    ),
    mesh=_mesh,
    compiler_params=pltpu.CompilerParams(needs_layout_passes=False),
    scratch_types=[
        pltpu.VMEM((_N,), jnp.float32),
        pltpu.VMEM((_N,), jnp.float32),
        pltpu.VMEM((_N,), jnp.float32),
        pltpu.VMEM((_N,), jnp.int32),
        pltpu.VMEM((_QPW,), jnp.float32),
        pltpu.VMEM((_QPW,), jnp.float32),
        pltpu.VMEM((_QPW,), jnp.float32),
        pltpu.VMEM((_QPW,), jnp.int32),
        pltpu.VMEM((_QPW * 64,), jnp.int32),
        pltpu.VMEM((_QPW * _S,), jnp.int32),
        pltpu.VMEM((_QPW * _S,), jnp.float32),
        pltpu.VMEM((_QPW * _S,), jnp.float32),
        pltpu.VMEM((_QPW * _S,), jnp.float32),
    ],
)
def _phase1(xs_hbm, ys_hbm, zs_hbm, comp_hbm, nxs_hbm, nys_hbm, nzs_hbm,
            ncomp_hbm, idx_out, gx_out, gy_out, gz_out,
            ptx, pty, ptz, ptc, qxv, qyv, qzv, qcv,
            st_idx, st_idxc, st_dx, st_dy, st_dz):
    w = _wid()
    b = w // _WPB
    q0 = b * _P + (w % _WPB) * _QPW
    pltpu.sync_copy(xs_hbm.at[pl.ds(b * _N, _N)], ptx)
    pltpu.sync_copy(ys_hbm.at[pl.ds(b * _N, _N)], pty)
    pltpu.sync_copy(zs_hbm.at[pl.ds(b * _N, _N)], ptz)
    pltpu.sync_copy(comp_hbm.at[pl.ds(b * _N, _N)], ptc)
    pltpu.sync_copy(nxs_hbm.at[pl.ds(q0, _QPW)], qxv)
    pltpu.sync_copy(nys_hbm.at[pl.ds(q0, _QPW)], qyv)
    pltpu.sync_copy(nzs_hbm.at[pl.ds(q0, _QPW)], qzv)
    pltpu.sync_copy(ncomp_hbm.at[pl.ds(q0, _QPW)], qcv)

    lane = lax.iota(jnp.int32, 16)
    zeros16 = jnp.zeros((16,), jnp.int32)
    unroll = 4
    smax = jnp.full((16,), _S, jnp.int32)

    qpack = 4
    stride = _QPW // qpack

    def per_query(p, carry):
        qs = tuple(p + i * stride for i in range(qpack))
        qxyzc = []
        bases = []
        for q in qs:
            spl = jnp.full((16,), q, jnp.int32)
            qxyzc.append((plsc.load_gather(qxv, [spl]),
                          plsc.load_gather(qyv, [spl]),
                          plsc.load_gather(qzv, [spl]),
                          plsc.load_gather(qcv, [spl])))
            sbase = q * 64
            bases.append((q * _S, jnp.full((16,), sbase, jnp.int32),
                          jnp.full((16,), sbase - 1, jnp.int32), sbase))
            st_idx[pl.ds(sbase, 16)] = zeros16

        def chunk(j, ccs):
            out = []
            for u in range(unroll):
                off = j * (16 * unroll) + u * 16
                px = ptx[pl.ds(off, 16)]
                py = pty[pl.ds(off, 16)]
                pz = ptz[pl.ds(off, 16)]
                pc = ptc[pl.ds(off, 16)]
                gidx = lane + off
                out = []
                for (qx, qy, qz, qc), (_, base_v, bm1_v, _sb), cc in zip(
                        qxyzc, bases, ccs):
                    dx = px - qx
                    dy = py - qy
                    dz = pz - qz
                    d2 = dx * dx + dy * dy + dz * dz
                    thr = jnp.where(pc == qc, _R2, _R2P).astype(jnp.float32)
                    elig = d2 < thr
                    incl = plsc.cumsum(elig.astype(jnp.int32))
                    pos = (bm1_v + cc) + incl
                    plsc.store_scatter(st_idx, [pos], gidx, mask=elig)
                    cc = jnp.minimum(
                        cc + plsc.all_reduce_population_count(elig), smax)
                    out.append(cc)
                ccs = tuple(out)
            return ccs

        ccs = lax.fori_loop(0, _N // (16 * unroll), chunk,
                            (zeros16,) * qpack)
        for (base, base_v, _b1, sbase), (qx, qy, qz, qc), cntc in zip(
                bases, qxyzc, ccs):
            pad = plsc.load_gather(st_idx, [base_v])
            for h in range(2):
                lids = lane + h * 16
                iv = jnp.where(lids >= cntc,
                               pad, st_idx[pl.ds(sbase + h * 16, 16)])
                st_idxc[pl.ds(base + h * 16, 16)] = iv
                st_dx[pl.ds(base + h * 16, 16)] = plsc.load_gather(ptx, [iv]) - qx
                st_dy[pl.ds(base + h * 16, 16)] = plsc.load_gather(pty, [iv]) - qy
                st_dz[pl.ds(base + h * 16, 16)] = plsc.load_gather(ptz, [iv]) - qz
        return carry

    lax.fori_loop(0, stride, per_query, 0)

    o0 = q0 * _S
    pltpu.sync_copy(st_idxc, idx_out.at[pl.ds(o0, _QPW * _S)])
    pltpu.sync_copy(st_dx, gx_out.at[pl.ds(o0, _QPW * _S)])
    pltpu.sync_copy(st_dy, gy_out.at[pl.ds(o0, _QPW * _S)])
    pltpu.sync_copy(st_dz, gz_out.at[pl.ds(o0, _QPW * _S)])


@functools.partial(
    pl.kernel,
    out_type=jax.ShapeDtypeStruct((_B * _C * _P * _S,), jnp.float32),
    mesh=_mesh,
    compiler_params=pltpu.CompilerParams(needs_layout_passes=False),
    scratch_types=[
        pltpu.VMEM((_WPC * _N,), jnp.float32),
        pltpu.VMEM((_CH,), jnp.int32),
        pltpu.VMEM((_WPC * _CH,), jnp.float32),
    ],
)
def _phase2(feat_hbm, idx_hbm, gf_out, tabs, idxc, stg):
    w = _wid()
    b = w // _WPB
    c0 = (w % _WPB) * _WPC
    pltpu.sync_copy(feat_hbm.at[pl.ds((b * _C + c0) * _N, _WPC * _N)], tabs)
    chbase = [jnp.full((16,), ch * _N, jnp.int32) for ch in range(_WPC)]

    def per_chunk(k, carry):
        k0 = k * _CH
        pltpu.sync_copy(idx_hbm.at[pl.ds(b * _P * _S + k0, _CH)], idxc)

        def inner(t, c2):
            for u in range(4):
                o = t * 64 + u * 16
                iv = idxc[pl.ds(o, 16)]
                for ch in range(_WPC):
                    stg[pl.ds(ch * _CH + o, 16)] = plsc.load_gather(
                        tabs, [chbase[ch] + iv])
            return c2

        lax.fori_loop(0, _CH // 64, inner, 0)
        for ch in range(_WPC):
            pltpu.sync_copy(
                stg.at[pl.ds(ch * _CH, _CH)],
                gf_out.at[pl.ds((b * _C + c0 + ch) * (_P * _S) + k0, _CH)])
        return carry

    lax.fori_loop(0, (_P * _S) // _CH, per_chunk, 0)


def kernel(xyz, new_xyz, components, new_components, features):
    xyz32 = xyz.astype(jnp.float32)
    nxyz32 = new_xyz.astype(jnp.float32)
    xs = xyz32[:, :, 0].reshape(-1)
    ys = xyz32[:, :, 1].reshape(-1)
    zs = xyz32[:, :, 2].reshape(-1)
    nxs = nxyz32[:, :, 0].reshape(-1)
    nys = nxyz32[:, :, 1].reshape(-1)
    nzs = nxyz32[:, :, 2].reshape(-1)
    comp = components.reshape(-1).astype(jnp.int32)
    ncomp = new_components.reshape(-1).astype(jnp.int32)
    idx, gx, gy, gz = _phase1(xs, ys, zs, comp, nxs, nys, nzs, ncomp)
    gfeat = _phase2(features.astype(jnp.float32).reshape(-1), idx)
    gxyz = jnp.stack([gx.reshape(_B, _P, _S), gy.reshape(_B, _P, _S),
                      gz.reshape(_B, _P, _S)], axis=1)
    return jnp.concatenate(
        [gxyz, gfeat.reshape(_B, _C, _P, _S)], axis=1)


# speedup vs baseline: 339.3412x; 1.1961x over previous
import functools

import jax
import jax.numpy as jnp
from jax import lax
from jax.experimental import pallas as pl
from jax.experimental.pallas import tpu as pltpu
from jax.experimental.pallas import tpu_sc as plsc

_B, _N, _P, _C, _S = 4, 8192, 2048, 64, 32
_R2 = 0.2 * 0.2
_R2P = (0.5 * 0.2) ** 2
_NW = 32
_WPB = _NW // _B
_QPW = _P // _WPB
_WPC = _C // _WPB
_CH = 4096

_mesh = plsc.VectorSubcoreMesh(core_axis_name="c", subcore_axis_name="s")


def _wid():
    return lax.axis_index("s") * 2 + lax.axis_index("c")


@functools.partial(
    pl.kernel,
    out_type=(
        jax.ShapeDtypeStruct((_B * _P * _S,), jnp.int32),
        jax.ShapeDtypeStruct((_B * _P * _S,), jnp.float32),
        jax.ShapeDtypeStruct((_B * _P * _S,), jnp.float32),
        jax.ShapeDtypeStruct((_B * _P * _S,), jnp.float32),
    ),
    mesh=_mesh,
    compiler_params=pltpu.CompilerParams(needs_layout_passes=False),
    scratch_types=[
        pltpu.VMEM((_N,), jnp.float32),
        pltpu.VMEM((_N,), jnp.float32),
        pltpu.VMEM((_N,), jnp.float32),
        pltpu.VMEM((_N,), jnp.int32),
        pltpu.VMEM((_QPW,), jnp.float32),
        pltpu.VMEM((_QPW,), jnp.float32),
        pltpu.VMEM((_QPW,), jnp.float32),
        pltpu.VMEM((_QPW,), jnp.int32),
        pltpu.VMEM((_QPW * 64,), jnp.int32),
        pltpu.VMEM((_QPW * _S,), jnp.int32),
        pltpu.VMEM((_QPW * _S,), jnp.float32),
        pltpu.VMEM((_QPW * _S,), jnp.float32),
        pltpu.VMEM((_QPW * _S,), jnp.float32),
    ],
)
def _phase1(xs_hbm, ys_hbm, zs_hbm, comp_hbm, nxs_hbm, nys_hbm, nzs_hbm,
            ncomp_hbm, idx_out, gx_out, gy_out, gz_out,
            ptx, pty, ptz, ptc, qxv, qyv, qzv, qcv,
            st_idx, st_idxc, st_dx, st_dy, st_dz):
    w = _wid()
    b = w // _WPB
    q0 = b * _P + (w % _WPB) * _QPW
    pltpu.sync_copy(xs_hbm.at[pl.ds(b * _N, _N)], ptx)
    pltpu.sync_copy(ys_hbm.at[pl.ds(b * _N, _N)], pty)
    pltpu.sync_copy(zs_hbm.at[pl.ds(b * _N, _N)], ptz)
    pltpu.sync_copy(comp_hbm.at[pl.ds(b * _N, _N)], ptc)
    pltpu.sync_copy(nxs_hbm.at[pl.ds(q0, _QPW)], qxv)
    pltpu.sync_copy(nys_hbm.at[pl.ds(q0, _QPW)], qyv)
    pltpu.sync_copy(nzs_hbm.at[pl.ds(q0, _QPW)], qzv)
    pltpu.sync_copy(ncomp_hbm.at[pl.ds(q0, _QPW)], qcv)

    lane = lax.iota(jnp.int32, 16)
    zeros16 = jnp.zeros((16,), jnp.int32)
    unroll = 2
    qpack = 8
    stride = _QPW // qpack

    def per_query(p, carry):
        qs = tuple(p + i * stride for i in range(qpack))
        qxyzc = []
        ccs0 = []
        for q in qs:
            spl = jnp.full((16,), q, jnp.int32)
            sbase = q * 64
            qxyzc.append((plsc.load_gather(qxv, [spl]),
                          plsc.load_gather(qyv, [spl]),
                          plsc.load_gather(qzv, [spl]),
                          plsc.load_gather(qcv, [spl]),
                          jnp.full((16,), sbase + 31, jnp.int32)))
            ccs0.append(jnp.full((16,), sbase - 1, jnp.int32))
            st_idx[pl.ds(sbase, 16)] = zeros16

        def chunk(j, ccs):
            out = []
            for u in range(unroll):
                off = j * (16 * unroll) + u * 16
                px = ptx[pl.ds(off, 16)]
                py = pty[pl.ds(off, 16)]
                pz = ptz[pl.ds(off, 16)]
                pc = ptc[pl.ds(off, 16)]
                gidx = lane + off
                out = []
                for (qx, qy, qz, qc, cap_v), ccb in zip(qxyzc, ccs):
                    dx = px - qx
                    dy = py - qy
                    dz = pz - qz
                    d2 = dx * dx + dy * dy + dz * dz
                    thr = jnp.where(pc == qc, _R2, _R2P).astype(jnp.float32)
                    elig = d2 < thr
                    incl = plsc.cumsum(elig.astype(jnp.int32))
                    plsc.store_scatter(st_idx, [ccb + incl], gidx, mask=elig)
                    ccb = jnp.minimum(
                        ccb + plsc.all_reduce_population_count(elig), cap_v)
                    out.append(ccb)
                ccs = tuple(out)
            return ccs

        ccs = lax.fori_loop(0, _N // (16 * unroll), chunk, tuple(ccs0))
        for q, (qx, qy, qz, qc, _cap), ccb in zip(qs, qxyzc, ccs):
            base = q * _S
            sbase = q * 64
            cntc = ccb - jnp.full((16,), sbase - 1, jnp.int32)
            pad = plsc.load_gather(st_idx, [jnp.full((16,), sbase, jnp.int32)])
            for h in range(2):
                lids = lane + h * 16
                iv = jnp.where(lids >= cntc,
                               pad, st_idx[pl.ds(sbase + h * 16, 16)])
                st_idxc[pl.ds(base + h * 16, 16)] = iv
                st_dx[pl.ds(base + h * 16, 16)] = plsc.load_gather(ptx, [iv]) - qx
                st_dy[pl.ds(base + h * 16, 16)] = plsc.load_gather(pty, [iv]) - qy
                st_dz[pl.ds(base + h * 16, 16)] = plsc.load_gather(ptz, [iv]) - qz
        return carry

    lax.fori_loop(0, stride, per_query, 0)

    o0 = q0 * _S
    pltpu.sync_copy(st_idxc, idx_out.at[pl.ds(o0, _QPW * _S)])
    pltpu.sync_copy(st_dx, gx_out.at[pl.ds(o0, _QPW * _S)])
    pltpu.sync_copy(st_dy, gy_out.at[pl.ds(o0, _QPW * _S)])
    pltpu.sync_copy(st_dz, gz_out.at[pl.ds(o0, _QPW * _S)])


@functools.partial(
    pl.kernel,
    out_type=jax.ShapeDtypeStruct((_B * _C * _P * _S,), jnp.float32),
    mesh=_mesh,
    compiler_params=pltpu.CompilerParams(needs_layout_passes=False),
    scratch_types=[
        pltpu.VMEM((_WPC * _N,), jnp.float32),
        pltpu.VMEM((_CH,), jnp.int32),
        pltpu.VMEM((_WPC * _CH,), jnp.float32),
    ],
)
def _phase2(feat_hbm, idx_hbm, gf_out, tabs, idxc, stg):
    w = _wid()
    b = w // _WPB
    c0 = (w % _WPB) * _WPC
    pltpu.sync_copy(feat_hbm.at[pl.ds((b * _C + c0) * _N, _WPC * _N)], tabs)
    chbase = [jnp.full((16,), ch * _N, jnp.int32) for ch in range(_WPC)]

    def per_chunk(k, carry):
        k0 = k * _CH
        pltpu.sync_copy(idx_hbm.at[pl.ds(b * _P * _S + k0, _CH)], idxc)

        def inner(t, c2):
            for u in range(4):
                o = t * 64 + u * 16
                iv = idxc[pl.ds(o, 16)]
                for ch in range(_WPC):
                    stg[pl.ds(ch * _CH + o, 16)] = plsc.load_gather(
                        tabs, [chbase[ch] + iv])
            return c2

        lax.fori_loop(0, _CH // 64, inner, 0)
        for ch in range(_WPC):
            pltpu.sync_copy(
                stg.at[pl.ds(ch * _CH, _CH)],
                gf_out.at[pl.ds((b * _C + c0 + ch) * (_P * _S) + k0, _CH)])
        return carry

    lax.fori_loop(0, (_P * _S) // _CH, per_chunk, 0)


def kernel(xyz, new_xyz, components, new_components, features):
    xyz32 = xyz.astype(jnp.float32)
    nxyz32 = new_xyz.astype(jnp.float32)
    xs = xyz32[:, :, 0].reshape(-1)
    ys = xyz32[:, :, 1].reshape(-1)
    zs = xyz32[:, :, 2].reshape(-1)
    nxs = nxyz32[:, :, 0].reshape(-1)
    nys = nxyz32[:, :, 1].reshape(-1)
    nzs = nxyz32[:, :, 2].reshape(-1)
    comp = components.reshape(-1).astype(jnp.int32)
    ncomp = new_components.reshape(-1).astype(jnp.int32)
    idx, gx, gy, gz = _phase1(xs, ys, zs, comp, nxs, nys, nzs, ncomp)
    gfeat = _phase2(features.astype(jnp.float32).reshape(-1), idx)
    gxyz = jnp.stack([gx.reshape(_B, _P, _S), gy.reshape(_B, _P, _S),
                      gz.reshape(_B, _P, _S)], axis=1)
    return jnp.concatenate(
        [gxyz, gfeat.reshape(_B, _C, _P, _S)], axis=1)
